# baseline (speedup 1.0000x reference)
import functools

import jax
import jax.numpy as jnp
from jax.experimental import pallas as pl
from jax.experimental.pallas import tpu as pltpu

LANE = 128
TM = 512
CHUNK = 512
SW = 256
VMEM_LIMIT_BYTES = 48 * 1024 * 1024


def _round_up(v, m):
    return (v + m - 1) // m * m


def _pad2d(x, rows, cols):
    r, c = x.shape
    if r == rows and c == cols:
        return x
    return jnp.pad(x, ((0, rows - r), (0, cols - c)))


def _xw_kernel(x_ref, w_ref, dinv_ref, o_ref):
    xw = jnp.dot(x_ref[...].astype(jnp.bfloat16), w_ref[...],
                 preferred_element_type=jnp.float32)
    o_ref[...] = xw * dinv_ref[...]


def _scaled_xw(x, w_bf16, dinv_col, tm):
    n, f_in = x.shape
    h = w_bf16.shape[1]
    return pl.pallas_call(
        _xw_kernel,
        out_shape=jax.ShapeDtypeStruct((n, h), jnp.float32),
        grid=(n // tm,),
        in_specs=[
            pl.BlockSpec((tm, f_in), lambda i: (i, 0)),
            pl.BlockSpec((f_in, h), lambda i: (0, 0)),
            pl.BlockSpec((tm, 1), lambda i: (i, 0)),
        ],
        out_specs=pl.BlockSpec((tm, h), lambda i: (i, 0)),
        compiler_params=pltpu.CompilerParams(
            dimension_semantics=("parallel",),
            vmem_limit_bytes=VMEM_LIMIT_BYTES,
        ),
    )(x, w_bf16, dinv_col)


def _agg_kernel(off_ref, src_ref, dst_ref, zview_ref, zrows_ref, dinv_ref,
                needs_ref, b_ref, w2_ref, o_ref, g_ref, acc_ref,
                *, n_half, second_layer):
    i = pl.program_id(0)
    acc_ref[...] = jnp.zeros_like(acc_ref)

    c_lo = off_ref[i] // CHUNK
    c_hi = (off_ref[i + 1] + CHUNK - 1) // CHUNK

    row_base = i * TM
    row_iota = jax.lax.broadcasted_iota(jnp.int32, (TM, CHUNK), 0) + row_base

    def chunk_body(c, _):
        ebase = c * CHUNK

        def gather_block(j, _):
            gbase = j * 128
            for u in range(128):
                s = pl.multiple_of(src_ref[ebase + gbase + u], 2)
                d = pl.multiple_of(2 * (gbase + u), 2)
                g_ref[pl.ds(d, 2), :] = zview_ref[pl.ds(s, 2), :]
            return 0

        jax.lax.fori_loop(0, CHUNK // 128, gather_block, 0)
        dst_v = dst_ref[c]
        m_oh = (row_iota == dst_v).astype(jnp.bfloat16)
        g_lo = g_ref[::2, :].astype(jnp.bfloat16)
        acc_ref[:, 0:128] += jnp.dot(m_oh, g_lo,
                                     preferred_element_type=jnp.float32)
        for h in range(1, n_half):
            g_hi = g_ref[h::2, :].astype(jnp.bfloat16)
            acc_ref[:, 128 * h:128 * (h + 1)] += jnp.dot(
                m_oh, g_hi, preferred_element_type=jnp.float32)
        return 0

    jax.lax.fori_loop(c_lo, c_hi, chunk_body, 0)

    width = 128 * n_half
    agg = acc_ref[...] + needs_ref[...] * zrows_ref[:, 0:width]
    pre = agg * dinv_ref[...] + b_ref[...]
    if second_layer:
        o_ref[...] = pre
    else:
        hmat = jnp.maximum(pre, 0.0)
        z2 = jnp.dot(hmat.astype(jnp.bfloat16), w2_ref[...],
                     preferred_element_type=jnp.float32) * dinv_ref[...]
        f2 = w2_ref.shape[1]
        o_ref[:, 0:f2] = z2
        if f2 < o_ref.shape[1]:
            o_ref[:, f2:] = jnp.zeros_like(o_ref[:, f2:])


def _sparse_agg(off, src_scaled, dst_chunks, zview, zrows, dinv_col,
                needs_col, b_row, w2_bf16, n, width, out_width,
                second_layer):
    n_half = width // 128
    n_chunks = dst_chunks.shape[0]
    kernel_body = functools.partial(
        _agg_kernel, n_half=n_half, second_layer=second_layer)
    return pl.pallas_call(
        kernel_body,
        out_shape=jax.ShapeDtypeStruct((n, out_width), jnp.float32),
        grid_spec=pltpu.PrefetchScalarGridSpec(
            num_scalar_prefetch=2,
            grid=(n // TM,),
            in_specs=[
                pl.BlockSpec((n_chunks, 1, CHUNK), lambda i, *_: (0, 0, 0)),
                pl.BlockSpec((2 * n, 128), lambda i, *_: (0, 0)),
                pl.BlockSpec((TM, zrows.shape[1]), lambda i, *_: (i, 0)),
                pl.BlockSpec((TM, 1), lambda i, *_: (i, 0)),
                pl.BlockSpec((TM, 1), lambda i, *_: (i, 0)),
                pl.BlockSpec((1, out_width if second_layer else width),
                             lambda i, *_: (0, 0)),
                pl.BlockSpec(w2_bf16.shape, lambda i, *_: (0, 0)),
            ],
            out_specs=pl.BlockSpec((TM, out_width), lambda i, *_: (i, 0)),
            scratch_shapes=[
                pltpu.VMEM((2 * CHUNK, 128), jnp.float32),
                pltpu.VMEM((TM, width), jnp.float32),
            ],
        ),
        compiler_params=pltpu.CompilerParams(
            dimension_semantics=("parallel",),
            vmem_limit_bytes=VMEM_LIMIT_BYTES,
        ),
        cost_estimate=pl.CostEstimate(
            flops=2 * (src_scaled.shape[0] // 2) * width,
            transcendentals=0,
            bytes_accessed=8 * n * width + 4 * n * out_width),
    )(off, src_scaled, dst_chunks, zview, zrows, dinv_col, needs_col,
      b_row, w2_bf16)


def kernel(x, edge_index, w1, b1, w2, b2):
    n, f_in = x.shape
    hidden = w1.shape[1]
    f_out = w2.shape[1]
    src, dst = edge_index[0], edge_index[1]
    e = src.shape[0]

    n_pad = _round_up(n, TM)
    f_in_p = _round_up(f_in, LANE)
    h_p = _round_up(hidden, LANE)
    f_out_p = _round_up(f_out, LANE)

    is_self = (src == dst)
    diag_cnt = jnp.zeros((n,), jnp.int32).at[dst].add(is_self.astype(jnp.int32))
    in_cnt = jnp.zeros((n,), jnp.int32).at[dst].add(1)
    needs_loop = (diag_cnt == 0)
    deg = (in_cnt + needs_loop.astype(jnp.int32)).astype(jnp.float32)
    d_inv_sqrt = jnp.where(deg > 0, jax.lax.rsqrt(deg), 0.0)
    dinv_col = jnp.pad(d_inv_sqrt, (0, n_pad - n))[:, None]
    needs_col = jnp.pad(needs_loop.astype(jnp.float32), (0, n_pad - n))[:, None]

    e_pad = _round_up(e, CHUNK)
    n_tiles = n_pad // TM
    bucket = jax.lax.shift_right_logical(dst, TM.bit_length() - 1)
    oh = (bucket[:, None] == jnp.arange(n_tiles, dtype=jnp.int32)[None, :])
    within = jnp.cumsum(oh.astype(jnp.int32), axis=0)
    counts = within[-1]
    base = jnp.concatenate([jnp.zeros((1,), jnp.int32),
                            jnp.cumsum(counts)[:-1].astype(jnp.int32)])
    rank = jnp.take_along_axis(within, bucket[:, None], 1)[:, 0] - 1
    newpos = base[bucket] + rank
    dst_s = jnp.full((e_pad,), n_pad, jnp.int32).at[newpos].set(dst)
    src_s = jnp.zeros((e_pad,), jnp.int32).at[newpos].set(src)
    off = jnp.concatenate([base, jnp.full((1,), e, jnp.int32)])
    src_scaled = src_s * 2
    dst_chunks = dst_s.reshape(e_pad // CHUNK, 1, CHUNK)

    assert h_p <= SW and f_out_p <= SW

    x_p = _pad2d(x, n_pad, f_in_p)
    w1b = _pad2d(w1, f_in_p, SW).astype(jnp.bfloat16)
    w2b = _pad2d(w2, SW, f_out_p).astype(jnp.bfloat16)
    b1_row = jnp.pad(b1, (0, SW - hidden)).reshape(1, SW)
    b2_row = jnp.pad(b2, (0, f_out_p - f_out)).reshape(1, f_out_p)

    z1 = _scaled_xw(x_p, w1b, dinv_col, tm=1024 if n_pad % 1024 == 0 else TM)
    z1_view = z1.reshape(2 * n_pad, 128)
    z2 = _sparse_agg(off, src_scaled, dst_chunks, z1_view, z1, dinv_col,
                     needs_col, b1_row, w2b, n_pad, SW, SW,
                     second_layer=False)

    z2_view = z2.reshape(2 * n_pad, 128)
    out = _sparse_agg(off, src_scaled, dst_chunks, z2_view, z2, dinv_col,
                      needs_col, b2_row, w2b, n_pad, f_out_p, f_out_p,
                      second_layer=True)
    return out[:n, :f_out]


# speedup vs baseline: 2.2052x; 2.2052x over previous
import functools

import jax
import jax.numpy as jnp
from jax.experimental import pallas as pl
from jax.experimental.pallas import tpu as pltpu

LANE = 128
TM = 512
CHUNK = 512
SW = 256
VMEM_LIMIT_BYTES = 48 * 1024 * 1024


def _round_up(v, m):
    return (v + m - 1) // m * m


def _pad2d(x, rows, cols):
    r, c = x.shape
    if r == rows and c == cols:
        return x
    return jnp.pad(x, ((0, rows - r), (0, cols - c)))


def _xw_kernel(x_ref, w_ref, dinv_ref, o_ref):
    xw = jnp.dot(x_ref[...].astype(jnp.bfloat16), w_ref[...],
                 preferred_element_type=jnp.float32)
    o_ref[...] = xw * dinv_ref[...]


def _scaled_xw(x, w_bf16, dinv_col, tm):
    n, f_in = x.shape
    h = w_bf16.shape[1]
    return pl.pallas_call(
        _xw_kernel,
        out_shape=jax.ShapeDtypeStruct((n, h), jnp.float32),
        grid=(n // tm,),
        in_specs=[
            pl.BlockSpec((tm, f_in), lambda i: (i, 0)),
            pl.BlockSpec((f_in, h), lambda i: (0, 0)),
            pl.BlockSpec((tm, 1), lambda i: (i, 0)),
        ],
        out_specs=pl.BlockSpec((tm, h), lambda i: (i, 0)),
        compiler_params=pltpu.CompilerParams(
            dimension_semantics=("parallel",),
            vmem_limit_bytes=VMEM_LIMIT_BYTES,
        ),
    )(x, w_bf16, dinv_col)


def _agg_kernel(off_ref, src_ref, dst_ref, zview_ref, zrows_ref, dinv_ref,
                needs_ref, b_ref, w2_ref, o_ref, g_ref, acc_ref,
                *, n_half, second_layer):
    i = pl.program_id(0)
    acc_ref[...] = jnp.zeros_like(acc_ref)

    c_lo = off_ref[i] // CHUNK
    c_hi = (off_ref[i + 1] + CHUNK - 1) // CHUNK

    row_base = i * TM
    row_iota = jax.lax.broadcasted_iota(jnp.int32, (TM, CHUNK), 0) + row_base

    def chunk_body(c, _):
        ebase = c * CHUNK

        def gather_block(j, _):
            gbase = j * 128
            for u in range(128):
                s = pl.multiple_of(src_ref[ebase + gbase + u], 2)
                d = pl.multiple_of(2 * (gbase + u), 2)
                g_ref[pl.ds(d, 2), :] = zview_ref[pl.ds(s, 2), :]
            return 0

        jax.lax.fori_loop(0, CHUNK // 128, gather_block, 0)
        dst_v = dst_ref[c]
        m_oh = (row_iota == dst_v).astype(jnp.bfloat16)
        g_lo = g_ref[::2, :].astype(jnp.bfloat16)
        acc_ref[:, 0:128] += jnp.dot(m_oh, g_lo,
                                     preferred_element_type=jnp.float32)
        for h in range(1, n_half):
            g_hi = g_ref[h::2, :].astype(jnp.bfloat16)
            acc_ref[:, 128 * h:128 * (h + 1)] += jnp.dot(
                m_oh, g_hi, preferred_element_type=jnp.float32)
        return 0

    jax.lax.fori_loop(c_lo, c_hi, chunk_body, 0)

    width = 128 * n_half
    agg = acc_ref[...] + needs_ref[...] * zrows_ref[:, 0:width]
    pre = agg * dinv_ref[...] + b_ref[...]
    if second_layer:
        o_ref[...] = pre
    else:
        hmat = jnp.maximum(pre, 0.0)
        z2 = jnp.dot(hmat.astype(jnp.bfloat16), w2_ref[...],
                     preferred_element_type=jnp.float32) * dinv_ref[...]
        f2 = w2_ref.shape[1]
        o_ref[:, 0:f2] = z2
        if f2 < o_ref.shape[1]:
            o_ref[:, f2:] = jnp.zeros_like(o_ref[:, f2:])


def _sparse_agg(off, src_scaled, dst_chunks, zview, zrows, dinv_col,
                needs_col, b_row, w2_bf16, n, width, out_width,
                second_layer):
    n_half = width // 128
    n_chunks = dst_chunks.shape[0]
    kernel_body = functools.partial(
        _agg_kernel, n_half=n_half, second_layer=second_layer)
    return pl.pallas_call(
        kernel_body,
        out_shape=jax.ShapeDtypeStruct((n, out_width), jnp.float32),
        grid_spec=pltpu.PrefetchScalarGridSpec(
            num_scalar_prefetch=2,
            grid=(n // TM,),
            in_specs=[
                pl.BlockSpec((n_chunks, 1, CHUNK), lambda i, *_: (0, 0, 0)),
                pl.BlockSpec((2 * n, 128), lambda i, *_: (0, 0)),
                pl.BlockSpec((TM, zrows.shape[1]), lambda i, *_: (i, 0)),
                pl.BlockSpec((TM, 1), lambda i, *_: (i, 0)),
                pl.BlockSpec((TM, 1), lambda i, *_: (i, 0)),
                pl.BlockSpec((1, out_width if second_layer else width),
                             lambda i, *_: (0, 0)),
                pl.BlockSpec(w2_bf16.shape, lambda i, *_: (0, 0)),
            ],
            out_specs=pl.BlockSpec((TM, out_width), lambda i, *_: (i, 0)),
            scratch_shapes=[
                pltpu.VMEM((2 * CHUNK, 128), jnp.float32),
                pltpu.VMEM((TM, width), jnp.float32),
            ],
        ),
        compiler_params=pltpu.CompilerParams(
            dimension_semantics=("parallel",),
            vmem_limit_bytes=VMEM_LIMIT_BYTES,
        ),
        cost_estimate=pl.CostEstimate(
            flops=2 * (src_scaled.shape[0] // 2) * width,
            transcendentals=0,
            bytes_accessed=8 * n * width + 4 * n * out_width),
    )(off, src_scaled, dst_chunks, zview, zrows, dinv_col, needs_col,
      b_row, w2_bf16)


def kernel(x, edge_index, w1, b1, w2, b2):
    n, f_in = x.shape
    hidden = w1.shape[1]
    f_out = w2.shape[1]
    src, dst = edge_index[0], edge_index[1]
    e = src.shape[0]

    n_pad = _round_up(n, TM)
    f_in_p = _round_up(f_in, LANE)
    h_p = _round_up(hidden, LANE)
    f_out_p = _round_up(f_out, LANE)

    is_self = (src == dst)
    diag_cnt = jnp.zeros((n,), jnp.int32).at[dst].add(is_self.astype(jnp.int32))
    in_cnt = jnp.zeros((n,), jnp.int32).at[dst].add(1)
    needs_loop = (diag_cnt == 0)
    deg = (in_cnt + needs_loop.astype(jnp.int32)).astype(jnp.float32)
    d_inv_sqrt = jnp.where(deg > 0, jax.lax.rsqrt(deg), 0.0)
    dinv_col = jnp.pad(d_inv_sqrt, (0, n_pad - n))[:, None]
    needs_col = jnp.pad(needs_loop.astype(jnp.float32), (0, n_pad - n))[:, None]

    e_pad = _round_up(e, CHUNK)
    node_bits = max(n_pad - 1, 1).bit_length()
    if 2 * node_bits <= 31:
        packed = jnp.sort((dst << node_bits) | src)
        dst_s = jax.lax.shift_right_logical(packed, node_bits)
        src_s = packed & ((1 << node_bits) - 1)
    else:
        dst_s, src_s = jax.lax.sort_key_val(dst, src)
    dst_s = jnp.concatenate(
        [dst_s, jnp.full((e_pad - e,), n_pad, jnp.int32)]) \
        if e_pad != e else dst_s
    src_s = jnp.concatenate(
        [src_s, jnp.zeros((e_pad - e,), jnp.int32)]) if e_pad != e else src_s
    off = jnp.searchsorted(dst_s, jnp.arange(0, n_pad + 1, TM,
                                             dtype=jnp.int32)).astype(jnp.int32)
    src_scaled = src_s * 2
    dst_chunks = dst_s.reshape(e_pad // CHUNK, 1, CHUNK)

    assert h_p <= SW and f_out_p <= SW

    x_p = _pad2d(x, n_pad, f_in_p)
    w1b = _pad2d(w1, f_in_p, SW).astype(jnp.bfloat16)
    w2b = _pad2d(w2, SW, f_out_p).astype(jnp.bfloat16)
    b1_row = jnp.pad(b1, (0, SW - hidden)).reshape(1, SW)
    b2_row = jnp.pad(b2, (0, f_out_p - f_out)).reshape(1, f_out_p)

    z1 = _scaled_xw(x_p, w1b, dinv_col, tm=1024 if n_pad % 1024 == 0 else TM)
    z1_view = z1.reshape(2 * n_pad, 128)
    z2 = _sparse_agg(off, src_scaled, dst_chunks, z1_view, z1, dinv_col,
                     needs_col, b1_row, w2b, n_pad, SW, SW,
                     second_layer=False)

    z2_view = z2.reshape(2 * n_pad, 128)
    out = _sparse_agg(off, src_scaled, dst_chunks, z2_view, z2, dinv_col,
                      needs_col, b2_row, w2b, n_pad, f_out_p, f_out_p,
                      second_layer=True)
    return out[:n, :f_out]


# speedup vs baseline: 2.2375x; 1.0146x over previous
import functools

import jax
import jax.numpy as jnp
from jax.experimental import pallas as pl
from jax.experimental.pallas import tpu as pltpu

LANE = 128
TM = 512
CHUNK = 1024
SW = 256
VMEM_LIMIT_BYTES = 48 * 1024 * 1024


def _round_up(v, m):
    return (v + m - 1) // m * m


def _pad2d(x, rows, cols):
    r, c = x.shape
    if r == rows and c == cols:
        return x
    return jnp.pad(x, ((0, rows - r), (0, cols - c)))


def _xw_kernel(x_ref, w_ref, dinv_ref, o_ref):
    xw = jnp.dot(x_ref[...].astype(jnp.bfloat16), w_ref[...],
                 preferred_element_type=jnp.float32)
    o_ref[...] = xw * dinv_ref[...]


def _scaled_xw(x, w_bf16, dinv_col, tm):
    n, f_in = x.shape
    h = w_bf16.shape[1]
    return pl.pallas_call(
        _xw_kernel,
        out_shape=jax.ShapeDtypeStruct((n, h), jnp.float32),
        grid=(n // tm,),
        in_specs=[
            pl.BlockSpec((tm, f_in), lambda i: (i, 0)),
            pl.BlockSpec((f_in, h), lambda i: (0, 0)),
            pl.BlockSpec((tm, 1), lambda i: (i, 0)),
        ],
        out_specs=pl.BlockSpec((tm, h), lambda i: (i, 0)),
        compiler_params=pltpu.CompilerParams(
            dimension_semantics=("parallel",),
            vmem_limit_bytes=VMEM_LIMIT_BYTES,
        ),
    )(x, w_bf16, dinv_col)


def _agg_kernel(off_ref, src_ref, dst_ref, zview_ref, zrows_ref, dinv_ref,
                needs_ref, b_ref, w2_ref, o_ref, g_ref, acc_ref,
                *, n_half, second_layer):
    i = pl.program_id(0)
    acc_ref[...] = jnp.zeros_like(acc_ref)

    c_lo = off_ref[i] // CHUNK
    c_hi = (off_ref[i + 1] + CHUNK - 1) // CHUNK

    row_base = i * TM
    row_iota = jax.lax.broadcasted_iota(jnp.int32, (TM, CHUNK), 0) + row_base

    def chunk_body(c, _):
        ebase = c * CHUNK

        def gather_block(j, _):
            gbase = j * 128
            for u in range(128):
                s = pl.multiple_of(src_ref[ebase + gbase + u], 2)
                d = pl.multiple_of(2 * (gbase + u), 2)
                g_ref[pl.ds(d, 2), :] = zview_ref[pl.ds(s, 2), :]
            return 0

        jax.lax.fori_loop(0, CHUNK // 128, gather_block, 0)
        dst_v = dst_ref[c]
        m_oh = (row_iota == dst_v).astype(jnp.bfloat16)
        g_lo = g_ref[::2, :].astype(jnp.bfloat16)
        acc_ref[:, 0:128] += jnp.dot(m_oh, g_lo,
                                     preferred_element_type=jnp.float32)
        for h in range(1, n_half):
            g_hi = g_ref[h::2, :].astype(jnp.bfloat16)
            acc_ref[:, 128 * h:128 * (h + 1)] += jnp.dot(
                m_oh, g_hi, preferred_element_type=jnp.float32)
        return 0

    jax.lax.fori_loop(c_lo, c_hi, chunk_body, 0)

    width = 128 * n_half
    agg = acc_ref[...] + needs_ref[...] * zrows_ref[:, 0:width]
    pre = agg * dinv_ref[...] + b_ref[...]
    if second_layer:
        o_ref[...] = pre
    else:
        hmat = jnp.maximum(pre, 0.0)
        z2 = jnp.dot(hmat.astype(jnp.bfloat16), w2_ref[...],
                     preferred_element_type=jnp.float32) * dinv_ref[...]
        f2 = w2_ref.shape[1]
        o_ref[:, 0:f2] = z2
        if f2 < o_ref.shape[1]:
            o_ref[:, f2:] = jnp.zeros_like(o_ref[:, f2:])


def _sparse_agg(off, src_scaled, dst_chunks, zview, zrows, dinv_col,
                needs_col, b_row, w2_bf16, n, width, out_width,
                second_layer):
    n_half = width // 128
    n_chunks = dst_chunks.shape[0]
    kernel_body = functools.partial(
        _agg_kernel, n_half=n_half, second_layer=second_layer)
    return pl.pallas_call(
        kernel_body,
        out_shape=jax.ShapeDtypeStruct((n, out_width), jnp.float32),
        grid_spec=pltpu.PrefetchScalarGridSpec(
            num_scalar_prefetch=2,
            grid=(n // TM,),
            in_specs=[
                pl.BlockSpec((n_chunks, 1, CHUNK), lambda i, *_: (0, 0, 0)),
                pl.BlockSpec((2 * n, 128), lambda i, *_: (0, 0)),
                pl.BlockSpec((TM, zrows.shape[1]), lambda i, *_: (i, 0)),
                pl.BlockSpec((TM, 1), lambda i, *_: (i, 0)),
                pl.BlockSpec((TM, 1), lambda i, *_: (i, 0)),
                pl.BlockSpec((1, out_width if second_layer else width),
                             lambda i, *_: (0, 0)),
                pl.BlockSpec(w2_bf16.shape, lambda i, *_: (0, 0)),
            ],
            out_specs=pl.BlockSpec((TM, out_width), lambda i, *_: (i, 0)),
            scratch_shapes=[
                pltpu.VMEM((2 * CHUNK, 128), jnp.float32),
                pltpu.VMEM((TM, width), jnp.float32),
            ],
        ),
        compiler_params=pltpu.CompilerParams(
            dimension_semantics=("parallel",),
            vmem_limit_bytes=VMEM_LIMIT_BYTES,
        ),
        cost_estimate=pl.CostEstimate(
            flops=2 * (src_scaled.shape[0] // 2) * width,
            transcendentals=0,
            bytes_accessed=8 * n * width + 4 * n * out_width),
    )(off, src_scaled, dst_chunks, zview, zrows, dinv_col, needs_col,
      b_row, w2_bf16)


def kernel(x, edge_index, w1, b1, w2, b2):
    n, f_in = x.shape
    hidden = w1.shape[1]
    f_out = w2.shape[1]
    src, dst = edge_index[0], edge_index[1]
    e = src.shape[0]

    n_pad = _round_up(n, TM)
    f_in_p = _round_up(f_in, LANE)
    h_p = _round_up(hidden, LANE)
    f_out_p = _round_up(f_out, LANE)

    is_self = (src == dst)
    diag_cnt = jnp.zeros((n,), jnp.int32).at[dst].add(is_self.astype(jnp.int32))
    in_cnt = jnp.zeros((n,), jnp.int32).at[dst].add(1)
    needs_loop = (diag_cnt == 0)
    deg = (in_cnt + needs_loop.astype(jnp.int32)).astype(jnp.float32)
    d_inv_sqrt = jnp.where(deg > 0, jax.lax.rsqrt(deg), 0.0)
    dinv_col = jnp.pad(d_inv_sqrt, (0, n_pad - n))[:, None]
    needs_col = jnp.pad(needs_loop.astype(jnp.float32), (0, n_pad - n))[:, None]

    e_pad = _round_up(e, CHUNK)
    node_bits = max(n_pad - 1, 1).bit_length()
    if 2 * node_bits <= 31:
        packed = jnp.sort((dst << node_bits) | src)
        dst_s = jax.lax.shift_right_logical(packed, node_bits)
        src_s = packed & ((1 << node_bits) - 1)
    else:
        dst_s, src_s = jax.lax.sort_key_val(dst, src)
    dst_s = jnp.concatenate(
        [dst_s, jnp.full((e_pad - e,), n_pad, jnp.int32)]) \
        if e_pad != e else dst_s
    src_s = jnp.concatenate(
        [src_s, jnp.zeros((e_pad - e,), jnp.int32)]) if e_pad != e else src_s
    off = jnp.searchsorted(dst_s, jnp.arange(0, n_pad + 1, TM,
                                             dtype=jnp.int32)).astype(jnp.int32)
    src_scaled = src_s * 2
    dst_chunks = dst_s.reshape(e_pad // CHUNK, 1, CHUNK)

    assert h_p <= SW and f_out_p <= SW

    x_p = _pad2d(x, n_pad, f_in_p)
    w1b = _pad2d(w1, f_in_p, SW).astype(jnp.bfloat16)
    w2b = _pad2d(w2, SW, f_out_p).astype(jnp.bfloat16)
    b1_row = jnp.pad(b1, (0, SW - hidden)).reshape(1, SW)
    b2_row = jnp.pad(b2, (0, f_out_p - f_out)).reshape(1, f_out_p)

    z1 = _scaled_xw(x_p, w1b, dinv_col, tm=1024 if n_pad % 1024 == 0 else TM)
    z1_view = z1.reshape(2 * n_pad, 128)
    z2 = _sparse_agg(off, src_scaled, dst_chunks, z1_view, z1, dinv_col,
                     needs_col, b1_row, w2b, n_pad, SW, SW,
                     second_layer=False)

    z2_view = z2.reshape(2 * n_pad, 128)
    out = _sparse_agg(off, src_scaled, dst_chunks, z2_view, z2, dinv_col,
                      needs_col, b2_row, w2b, n_pad, f_out_p, f_out_p,
                      second_layer=True)
    return out[:n, :f_out]
